# baseline (speedup 1.0000x reference)
import jax
import jax.numpy as jnp
from jax.experimental import pallas as pl

CB = 8


def _add_body(x_ref, p_ref, o_ref):
    o_ref[...] = x_ref[...] + p_ref[...]


def kernel(x, pos_table):
    B, M, D = x.shape
    return pl.pallas_call(
        _add_body,
        grid=(B // CB,),
        in_specs=[
            pl.BlockSpec((CB, M, D), lambda i: (i, 0, 0)),
            pl.BlockSpec((M, D), lambda i: (0, 0)),
        ],
        out_specs=pl.BlockSpec((CB, M, D), lambda i: (i, 0, 0)),
        out_shape=jax.ShapeDtypeStruct((B, M, D), x.dtype),
    )(x, pos_table)


# speedup vs baseline: 5.6864x; 5.6864x over previous
import jax
import jax.numpy as jnp
from jax.experimental import pallas as pl

CB = 8


def _add_body(x_ref, p_ref, o_ref):
    o_ref[...] = x_ref[...] + p_ref[...]


def kernel(x, pos_table):
    B, M, D = x.shape
    xt = jnp.transpose(x, (0, 2, 1))
    pt = jnp.transpose(pos_table, (1, 0))
    out_t = pl.pallas_call(
        _add_body,
        grid=(B // CB,),
        in_specs=[
            pl.BlockSpec((CB, D, M), lambda i: (i, 0, 0)),
            pl.BlockSpec((D, M), lambda i: (0, 0)),
        ],
        out_specs=pl.BlockSpec((CB, D, M), lambda i: (i, 0, 0)),
        out_shape=jax.ShapeDtypeStruct((B, D, M), x.dtype),
    )(xt, pt)
    return jnp.transpose(out_t, (0, 2, 1))


# speedup vs baseline: 6.2520x; 1.0995x over previous
import jax
import jax.numpy as jnp
from jax.experimental import pallas as pl

CB = 16


def _add_body(x_ref, p_ref, o_ref):
    o_ref[...] = x_ref[...] + p_ref[...]


def kernel(x, pos_table):
    B, M, D = x.shape
    xt = jnp.transpose(x, (0, 2, 1))
    pt = jnp.transpose(pos_table, (1, 0))
    out_t = pl.pallas_call(
        _add_body,
        grid=(B // CB,),
        in_specs=[
            pl.BlockSpec((CB, D, M), lambda i: (i, 0, 0)),
            pl.BlockSpec((D, M), lambda i: (0, 0)),
        ],
        out_specs=pl.BlockSpec((CB, D, M), lambda i: (i, 0, 0)),
        out_shape=jax.ShapeDtypeStruct((B, D, M), x.dtype),
    )(xt, pt)
    return jnp.transpose(out_t, (0, 2, 1))


# speedup vs baseline: 6.3232x; 1.0114x over previous
import jax
import jax.numpy as jnp
from jax.experimental import pallas as pl

CB = 32


def _add_body(x_ref, p_ref, o_ref):
    o_ref[...] = x_ref[...] + p_ref[...]


def kernel(x, pos_table):
    B, M, D = x.shape
    xt = jnp.transpose(x, (0, 2, 1))
    pt = jnp.transpose(pos_table, (1, 0))
    out_t = pl.pallas_call(
        _add_body,
        grid=(B // CB,),
        in_specs=[
            pl.BlockSpec((CB, D, M), lambda i: (i, 0, 0)),
            pl.BlockSpec((D, M), lambda i: (0, 0)),
        ],
        out_specs=pl.BlockSpec((CB, D, M), lambda i: (i, 0, 0)),
        out_shape=jax.ShapeDtypeStruct((B, D, M), x.dtype),
    )(xt, pt)
    return jnp.transpose(out_t, (0, 2, 1))
